# baseline (speedup 1.0000x reference)
import functools

import jax
import jax.numpy as jnp
from jax import lax
from jax.experimental import pallas as pl
from jax.experimental.pallas import tpu as pltpu
from jax.experimental.pallas import tpu_sc as plsc

N_NODES = 10000
N_EDGES = 320000
D_N = 128
D_OUT = 16

NUM_CORES = 2
NUM_SUBCORES = 16
NUM_WORKERS = NUM_CORES * NUM_SUBCORES
EDGES_PER_WORKER = N_EDGES // NUM_WORKERS
CHUNK = 1000
SROWS = CHUNK // 8
NUM_CHUNKS = EDGES_PER_WORKER // CHUNK
NUM_PAIRS = NUM_CHUNKS // 2


def _proj_body(x_ref, wc_ref, brow_ref, p_ref):
  p_ref[...] = (
      jnp.dot(x_ref[...], wc_ref[...], preferred_element_type=jnp.float32)
      + brow_ref[...]
  )


def _node_proj(x, wc_pad, bias_row):
  grid = 5
  rows = N_NODES // grid
  return pl.pallas_call(
      _proj_body,
      grid=(grid,),
      in_specs=[
          pl.BlockSpec((rows, D_N), lambda i: (i, 0)),
          pl.BlockSpec((D_N, 128), lambda i: (0, 0)),
          pl.BlockSpec((1, 128), lambda i: (0, 0)),
      ],
      out_specs=pl.BlockSpec((rows, 128), lambda i: (i, 0)),
      out_shape=jax.ShapeDtypeStruct((N_NODES, 128), jnp.float32),
  )(x, wc_pad, bias_row)


def _self_body(ev_ref, wblk_ref, s_ref):
  s_ref[...] = jnp.dot(
      ev_ref[...], wblk_ref[...], preferred_element_type=jnp.float32
  )


def _self_map(ev_packed, w_blk):
  grid = 10
  rows = ev_packed.shape[0] // grid
  return pl.pallas_call(
      _self_body,
      grid=(grid,),
      in_specs=[
          pl.BlockSpec((rows, 128), lambda i: (i, 0)),
          pl.BlockSpec((128, 128), lambda i: (0, 0)),
      ],
      out_specs=pl.BlockSpec((rows, 128), lambda i: (i, 0)),
      out_shape=jax.ShapeDtypeStruct(ev_packed.shape, jnp.float32),
  )(ev_packed, w_blk)


_MESH = plsc.VectorSubcoreMesh(core_axis_name="c", subcore_axis_name="s")


@functools.partial(
    pl.kernel,
    out_type=jax.ShapeDtypeStruct((N_EDGES, D_OUT), jnp.float32),
    mesh=_MESH,
    scratch_types=[
        pltpu.VMEM((EDGES_PER_WORKER,), jnp.int32),
        pltpu.VMEM((EDGES_PER_WORKER,), jnp.int32),
        pltpu.VMEM((2, CHUNK, D_OUT), jnp.float32),
        pltpu.VMEM((2, CHUNK, D_OUT), jnp.float32),
        pltpu.VMEM((2, SROWS, 128), jnp.float32),
        pltpu.SemaphoreType.DMA,
        pltpu.SemaphoreType.DMA,
        pltpu.SemaphoreType.DMA,
        pltpu.SemaphoreType.DMA,
        pltpu.SemaphoreType.DMA,
        pltpu.SemaphoreType.DMA,
    ],
    compiler_params=pltpu.CompilerParams(use_tc_tiling_on_sc=False),
)
def _sc_gather_sum(p12_hbm, ei_hbm, s_hbm, g_hbm,
                   si_v, di_v, r1_v, r2_v, s_v,
                   semg0, semg1, sems0, sems1, semo0, semo1):
  wid = lax.axis_index("s") * NUM_CORES + lax.axis_index("c")
  base = wid * EDGES_PER_WORKER
  srow_base = wid * (EDGES_PER_WORKER // 8)
  semg = (semg0, semg1)
  sems = (sems0, sems1)
  semo = (semo0, semo1)

  pltpu.sync_copy(ei_hbm.at[0, pl.ds(base, EDGES_PER_WORKER)], si_v)
  pltpu.sync_copy(ei_hbm.at[1, pl.ds(base, EDGES_PER_WORKER)], di_v)

  @plsc.parallel_loop(0, EDGES_PER_WORKER // 16, unroll=8)
  def _(i):
    sl = pl.ds(i * 16, 16)
    si_v[sl] = si_v[sl] * 8
    di_v[sl] = di_v[sl] * 8 + 1

  def issue(c, b):
    sl = pl.ds(c * CHUNK, CHUNK)
    pltpu.async_copy(p12_hbm.at[si_v.at[sl]], r1_v.at[b], semg[b])
    pltpu.async_copy(p12_hbm.at[di_v.at[sl]], r2_v.at[b], semg[b])
    pltpu.async_copy(
        s_hbm.at[pl.ds(srow_base + c * SROWS, SROWS)], s_v.at[b], sems[b]
    )

  def wait_in(b):
    g = pltpu.make_async_copy(
        p12_hbm.at[si_v.at[pl.ds(0, CHUNK)]], r1_v.at[b], semg[b]
    )
    g.wait()
    g.wait()
    pltpu.make_async_copy(
        s_hbm.at[pl.ds(0, SROWS)], s_v.at[b], sems[b]
    ).wait()

  def wait_out(b):
    pltpu.make_async_copy(
        r1_v.at[b], g_hbm.at[pl.ds(0, CHUNK)], semo[b]
    ).wait()

  def store_out(c, b):
    pltpu.async_copy(
        r1_v.at[b], g_hbm.at[pl.ds(base + c * CHUNK, CHUNK)], semo[b]
    )

  def compute(b):
    r1_b = r1_v.at[b]
    r2_b = r2_v.at[b]
    s_b = s_v.at[b]

    @plsc.parallel_loop(0, SROWS, unroll=2)
    def _(r):
      e0 = r * 8
      for j in range(8):
        lanes = pl.ds(j * D_OUT, D_OUT)
        v = r1_b[e0 + j, :] + r2_b[e0 + j, :] + s_b[r, lanes]
        r1_b[e0 + j, :] = jnp.maximum(v, 0.0)

  def process(c, b, k):
    wait_in(b)

    @pl.when(k > 0)
    def _():
      wait_out(b)

    compute(b)
    store_out(c, b)

    @pl.when(c + 2 < NUM_CHUNKS)
    def _():
      issue(c + 2, b)

  issue(0, 0)
  issue(1, 1)

  def pair_body(k, carry):
    process(2 * k, 0, k)
    process(2 * k + 1, 1, k)
    return carry

  lax.fori_loop(0, NUM_PAIRS, pair_body, 0)

  wait_out(0)
  wait_out(1)


def kernel(X, edge_index, edge_vals, W_pass, b_pass, W_self, b_self):
  wa = W_pass[:, :D_N]
  wb = W_pass[:, D_N:]
  wc1 = ((wa + wb) * 0.5).T
  wc2 = ((wb - wa) * 0.5).T
  wc_pad = jnp.zeros((D_N, 128), jnp.float32)
  wc_pad = wc_pad.at[:, :D_OUT].set(wc1).at[:, D_OUT : 2 * D_OUT].set(wc2)
  bias_row = jnp.zeros((1, 128), jnp.float32)
  bias_row = bias_row.at[0, :D_OUT].set(b_pass + b_self)

  w_blk = jnp.kron(jnp.eye(8, dtype=jnp.float32), W_self.T)
  ev_packed = edge_vals.reshape(N_EDGES // 8, 128)

  p12 = _node_proj(X, wc_pad, bias_row)
  s_packed = _self_map(ev_packed, w_blk)
  p12_rows = p12.reshape(N_NODES * 8, D_OUT)

  return _sc_gather_sum(p12_rows, edge_index.astype(jnp.int32), s_packed)


# speedup vs baseline: 1.4730x; 1.4730x over previous
import functools

import jax
import jax.numpy as jnp
from jax import lax
from jax.experimental import pallas as pl
from jax.experimental.pallas import tpu as pltpu
from jax.experimental.pallas import tpu_sc as plsc

N_NODES = 10000
N_EDGES = 320000
D_N = 128
D_OUT = 16

NUM_CORES = 2
NUM_SUBCORES = 16
NUM_WORKERS = NUM_CORES * NUM_SUBCORES
EDGES_PER_WORKER = N_EDGES // NUM_WORKERS
CHUNK = 1000
NUM_CHUNKS = EDGES_PER_WORKER // CHUNK
NUM_PAIRS = NUM_CHUNKS // 2


def _proj_body(x_ref, wc_ref, brow_ref, p_ref):
  p_ref[...] = (
      jnp.dot(x_ref[...], wc_ref[...], preferred_element_type=jnp.float32)
      + brow_ref[...]
  )


def _node_proj(x, wc_pad, bias_row):
  grid = 5
  rows = N_NODES // grid
  return pl.pallas_call(
      _proj_body,
      grid=(grid,),
      in_specs=[
          pl.BlockSpec((rows, D_N), lambda i: (i, 0)),
          pl.BlockSpec((D_N, 128), lambda i: (0, 0)),
          pl.BlockSpec((1, 128), lambda i: (0, 0)),
      ],
      out_specs=pl.BlockSpec((rows, 128), lambda i: (i, 0)),
      out_shape=jax.ShapeDtypeStruct((N_NODES, 128), jnp.float32),
  )(x, wc_pad, bias_row)


def _epi_body(gt_ref, evt_ref, w_ref, ot_ref):
  st = lax.dot_general(
      w_ref[...], evt_ref[...], (((1,), (0,)), ((), ())),
      preferred_element_type=jnp.float32,
  )
  ot_ref[...] = jnp.maximum(gt_ref[...] + st, 0.0)


def _epilogue(g_t, ev_t, w_self):
  grid = 10
  cols = N_EDGES // grid
  return pl.pallas_call(
      _epi_body,
      grid=(grid,),
      in_specs=[
          pl.BlockSpec((D_OUT, cols), lambda i: (0, i)),
          pl.BlockSpec((D_OUT, cols), lambda i: (0, i)),
          pl.BlockSpec((D_OUT, D_OUT), lambda i: (0, 0)),
      ],
      out_specs=pl.BlockSpec((D_OUT, cols), lambda i: (0, i)),
      out_shape=jax.ShapeDtypeStruct((D_OUT, N_EDGES), jnp.float32),
  )(g_t, ev_t, w_self)


_MESH = plsc.VectorSubcoreMesh(core_axis_name="c", subcore_axis_name="s")


@functools.partial(
    pl.kernel,
    out_type=jax.ShapeDtypeStruct((N_EDGES, D_OUT), jnp.float32),
    mesh=_MESH,
    scratch_types=[
        pltpu.VMEM((EDGES_PER_WORKER,), jnp.int32),
        pltpu.VMEM((EDGES_PER_WORKER,), jnp.int32),
        pltpu.VMEM((2, CHUNK, D_OUT), jnp.float32),
        pltpu.VMEM((2, CHUNK, D_OUT), jnp.float32),
        pltpu.SemaphoreType.DMA,
        pltpu.SemaphoreType.DMA,
        pltpu.SemaphoreType.DMA,
        pltpu.SemaphoreType.DMA,
    ],
    compiler_params=pltpu.CompilerParams(use_tc_tiling_on_sc=False),
)
def _sc_gather_sum(p12_hbm, ei_hbm, g_hbm,
                   si_v, di_v, r1_v, r2_v,
                   semg0, semg1, semo0, semo1):
  wid = lax.axis_index("s") * NUM_CORES + lax.axis_index("c")
  base = wid * EDGES_PER_WORKER
  semg = (semg0, semg1)
  semo = (semo0, semo1)

  pltpu.sync_copy(ei_hbm.at[0, pl.ds(base, EDGES_PER_WORKER)], si_v)
  pltpu.sync_copy(ei_hbm.at[1, pl.ds(base, EDGES_PER_WORKER)], di_v)

  @plsc.parallel_loop(0, EDGES_PER_WORKER // 16, unroll=8)
  def _(i):
    sl = pl.ds(i * 16, 16)
    si_v[sl] = si_v[sl] * 8
    di_v[sl] = di_v[sl] * 8 + 1

  def issue(c, b):
    sl = pl.ds(c * CHUNK, CHUNK)
    pltpu.async_copy(p12_hbm.at[si_v.at[sl]], r1_v.at[b], semg[b])
    pltpu.async_copy(p12_hbm.at[di_v.at[sl]], r2_v.at[b], semg[b])

  def wait_in(b):
    g = pltpu.make_async_copy(
        p12_hbm.at[si_v.at[pl.ds(0, CHUNK)]], r1_v.at[b], semg[b]
    )
    g.wait()
    g.wait()

  def wait_out(b):
    pltpu.make_async_copy(
        r1_v.at[b], g_hbm.at[pl.ds(0, CHUNK)], semo[b]
    ).wait()

  def store_out(c, b):
    pltpu.async_copy(
        r1_v.at[b], g_hbm.at[pl.ds(base + c * CHUNK, CHUNK)], semo[b]
    )

  def compute(b):
    r1_b = r1_v.at[b]
    r2_b = r2_v.at[b]

    @plsc.parallel_loop(0, CHUNK, unroll=8)
    def _(e):
      r1_b[e, :] = r1_b[e, :] + r2_b[e, :]

  def process(c, b, k):
    wait_in(b)

    @pl.when(k > 0)
    def _():
      wait_out(b)

    compute(b)
    store_out(c, b)

    @pl.when(c + 2 < NUM_CHUNKS)
    def _():
      issue(c + 2, b)

  issue(0, 0)
  issue(1, 1)

  def pair_body(k, carry):
    process(2 * k, 0, k)
    process(2 * k + 1, 1, k)
    return carry

  lax.fori_loop(0, NUM_PAIRS, pair_body, 0)

  wait_out(0)
  wait_out(1)


def kernel(X, edge_index, edge_vals, W_pass, b_pass, W_self, b_self):
  wa = W_pass[:, :D_N]
  wb = W_pass[:, D_N:]
  wc1 = ((wa + wb) * 0.5).T
  wc2 = ((wb - wa) * 0.5).T
  wc_pad = jnp.zeros((D_N, 128), jnp.float32)
  wc_pad = wc_pad.at[:, :D_OUT].set(wc1).at[:, D_OUT : 2 * D_OUT].set(wc2)
  bias_row = jnp.zeros((1, 128), jnp.float32)
  bias_row = bias_row.at[0, :D_OUT].set(b_pass + b_self)

  p12 = _node_proj(X, wc_pad, bias_row)
  p12_rows = p12.reshape(N_NODES * 8, D_OUT)

  g = _sc_gather_sum(p12_rows, edge_index.astype(jnp.int32))
  out_t = _epilogue(g.T, edge_vals.T, W_self)
  return out_t.T
